# baseline (speedup 1.0000x reference)
import jax
import jax.numpy as jnp


@jax.jit
def kernel(x, centroids, temperature):
    cn = centroids / jnp.maximum(jnp.linalg.norm(centroids, axis=1, keepdims=True), 1e-12)
    logits = x @ cn.T
    inv = 1.0 / jnp.maximum(jnp.linalg.norm(x, axis=1, keepdims=True), 1e-12)
    return logits * inv / temperature


# speedup vs baseline: 1.3173x; 1.3173x over previous
import jax
import jax.numpy as jnp
from jax.experimental import pallas as pl
from jax.experimental.pallas import tpu as pltpu

_TOKENS = 32768
_DIM = 768
_EXPERTS = 64
_BT = 8192


def _router_kernel(x_ref, c_ref, t_ref, out_ref, cn_ref):
    @pl.when(pl.program_id(0) == 0)
    def _init():
        c = c_ref[:]
        c_ss = jnp.sum(c * c, axis=1, keepdims=True)
        cn_ref[:] = c * jax.lax.rsqrt(jnp.maximum(c_ss, 1e-24))

    xb = x_ref[:]
    x_ss = jnp.sum(xb * xb, axis=1, keepdims=True)
    inv_norm = jax.lax.rsqrt(jnp.maximum(x_ss, 1e-24))
    logits = jax.lax.dot_general(
        xb, cn_ref[:], (((1,), (1,)), ((), ())), preferred_element_type=jnp.float32
    )
    out_ref[:] = logits * (inv_norm / t_ref[0])


@jax.jit
def kernel(x, centroids, temperature):
    grid = (_TOKENS // _BT,)
    return pl.pallas_call(
        _router_kernel,
        grid=grid,
        in_specs=[
            pl.BlockSpec((_BT, _DIM), lambda i: (i, 0)),
            pl.BlockSpec((_EXPERTS, _DIM), lambda i: (0, 0)),
            pl.BlockSpec(memory_space=pltpu.SMEM),
        ],
        out_specs=pl.BlockSpec((_BT, _EXPERTS), lambda i: (i, 0)),
        out_shape=jax.ShapeDtypeStruct((_TOKENS, _EXPERTS), jnp.float32),
        scratch_shapes=[pltpu.VMEM((_EXPERTS, _DIM), jnp.float32)],
        compiler_params=pltpu.CompilerParams(
            dimension_semantics=("arbitrary",),
            vmem_limit_bytes=100 * 1024 * 1024,
        ),
    )(x, centroids, temperature)
